# baseline (speedup 1.0000x reference)
import functools

import jax
import jax.numpy as jnp
from jax import lax
from jax.experimental import pallas as pl
from jax.experimental.pallas import tpu as pltpu

K = 12


def _silu(z):
    return z * (1.0 / (1.0 + jnp.exp(-z)))


def _mm(a, b):
    return lax.dot_general(a, b, (((1,), (0,)), ((), ())),
                           preferred_element_type=jnp.float32)


def _mmT(a, b):
    return lax.dot_general(a, b, (((0,), (0,)), ((), ())),
                           preferred_element_type=jnp.float32)


def _body(NP, D, L,
          x3, xt, v3, sc, roh, t2, role_tbl,
          wt1, bt1, wt2, bt2, wsc, bsc, r1, br1, r2, br2,
          wm1ab, wsqzd, bm1, wm2, bm2,
          wcv1, bcv1, wc2t, bc2, wv2t, bv2,
          wn1, bn1, wn2, bn2,
          whs, bhs, whr, bhr,
          xo, vo, so, ro, idx_ref):
    f32 = jnp.float32
    xb = x3[0]
    xtb = xt[0]
    vb = v3[0]

    tval = t2[0, 0, 0]
    te = _silu(tval * wt1[...] + bt1[...])
    temb = _mm(te, wt2[...]) + bt2[...]
    scemb = _mm(sc[0], wsc[...]) + bsc[...]
    remb = _mm(roh[0], role_tbl[...])
    r1v = r1[...]
    pre = (_mm(scemb, r1v[0:D]) + _mm(remb, r1v[D:2 * D])
           + _mm(temb, r1v[2 * D:3 * D]) + br1[...])
    h0 = _mm(_silu(pre), r2[...]) + br2[...]

    colf = lax.broadcasted_iota(jnp.int32, (NP, NP), 1).astype(f32)
    rowf = lax.broadcasted_iota(jnp.int32, (NP, NP), 0).astype(f32)
    d2 = (((xb[:, 0:1] - xtb[0:1, :]) ** 2
           + (xb[:, 1:2] - xtb[1:2, :]) ** 2)
          + (xb[:, 2:3] - xtb[2:3, :]) ** 2)
    d2 = jnp.where(rowf == colf, 1e10, d2)
    for k in range(K):
        m = jnp.min(d2, axis=1, keepdims=True)
        idx = jnp.min(jnp.where(d2 == m, colf, f32(NP)), axis=1,
                      keepdims=True)
        idx_ref[k] = idx
        d2 = jnp.where(colf == idx, 1e10, d2)

    zpad = jnp.zeros((NP, D - 3), f32)
    zpad6 = jnp.zeros((NP, D - 6), f32)

    def layer(l, hxv):
        h, xc, vc = hxv
        hw = _mm(h, wm1ab[l])
        base = hw[:, D:2 * D] + bm1[l]
        gsrc = jnp.concatenate([hw[:, 0:D], xc, zpad], axis=1)
        wsq = wsqzd[l, 0:1]
        wzd = wsqzd[l, 1:2]
        wm2l, bm2l = wm2[l], bm2[l]
        wcv1l, bcv1l = wcv1[l], bcv1[l]
        wc2l, bc2l = wc2t[l], bc2[l]
        wv2l, bv2l = wv2t[l], bv2[l]

        def edge_slot(k, acc):
            idx = idx_ref[k]
            sel = (colf == idx).astype(f32)
            g = _mm(sel, gsrc)
            rel = g[:, D:D + 3] - xc
            sq = jnp.sum(rel * rel, axis=1, keepdims=True)
            zd = rel[:, 2:3]
            pre1 = g[:, 0:D] + base + sq * wsq + zd * wzd
            msg = _silu(_mm(_silu(pre1), wm2l) + bm2l)
            tcv = _silu(_mm(msg, wcv1l) + bcv1l)
            xw = (jnp.sum(tcv[:, 0:D] * wc2l, axis=1, keepdims=True)
                  + bc2l) / (sq + 1e-8)
            vw = (jnp.sum(tcv[:, D:2 * D] * wv2l, axis=1, keepdims=True)
                  + bv2l) / (sq + 1e-8)
            eo = jnp.concatenate([msg, rel * xw, rel * vw, zpad6], axis=1)
            return acc + _mmT(sel, eo)

        acc = lax.fori_loop(0, K, edge_slot,
                            jnp.zeros((NP, 2 * D), f32))
        aggr = acc[:, 0:D]
        xn = xc + acc[:, D:D + 3]
        vn = vc + acc[:, D + 3:D + 6]
        hcat = jnp.concatenate([h, aggr], axis=1)
        hpre = _silu(_mm(hcat, wn1[l]) + bn1[l])
        hn = h + _mm(hpre, wn2[l]) + bn2[l]
        return hn, xn, vn

    h, xc, vc = lax.fori_loop(0, L, layer, (h0, xb, vb))

    xo[0] = xc
    vo[0] = vc
    so[0] = _mm(h, whs[...]) + bhs[...]
    ro[0] = _mm(h, whr[...]) + bhr[...]


def kernel(x, v, scalars, roles, t, batch, params):
    f32 = jnp.float32
    B = t.shape[0]
    NTOT = x.shape[0]
    NP = NTOT // B
    NR, D = params["role_emb"].shape
    NS = scalars.shape[1]
    lys = params["layers"]
    L = len(lys)

    def st_w(name):
        return jnp.stack([ly[name]["w"] for ly in lys])

    def st_b(name):
        return jnp.stack([ly[name]["b"] for ly in lys])[:, None, :]

    wm1 = st_w("msg1")
    wm1ab = jnp.concatenate([wm1[:, 0:D, :], wm1[:, D:2 * D, :]], axis=2)
    wsqzd = wm1[:, 2 * D:2 * D + 2, :]
    bm1 = st_b("msg1")
    wm2, bm2 = st_w("msg2"), st_b("msg2")
    wcv1 = jnp.concatenate([st_w("coord1"), st_w("vec1")], axis=2)
    bcv1 = jnp.concatenate([st_b("coord1"), st_b("vec1")], axis=2)
    wn1, bn1 = st_w("node1"), st_b("node1")
    wn2, bn2 = st_w("node2"), st_b("node2")
    wc2t = jnp.stack([ly["coord2"]["w"].T for ly in lys])
    bc2 = jnp.stack([ly["coord2"]["b"] for ly in lys])[:, :, None]
    wv2t = jnp.stack([ly["vec2"]["w"].T for ly in lys])
    bv2 = jnp.stack([ly["vec2"]["b"] for ly in lys])[:, :, None]

    xb = x.reshape(B, NP, 3)
    xt = xb.transpose(0, 2, 1)
    vb = v.reshape(B, NP, 3)
    scb = scalars.reshape(B, NP, NS)
    roh = (roles.reshape(B, NP)[..., None]
           == jnp.arange(NR, dtype=roles.dtype)).astype(f32)
    t2 = t.reshape(B, 1, 1)

    wt1 = params["time1"]["w"]
    bt1 = params["time1"]["b"].reshape(1, D)
    wt2 = params["time2"]["w"]
    bt2 = params["time2"]["b"].reshape(1, D)
    wsc = params["scalar_emb"]["w"]
    bsc = params["scalar_emb"]["b"].reshape(1, D)
    r1 = params["roots1"]["w"]
    br1 = params["roots1"]["b"].reshape(1, D)
    r2 = params["roots2"]["w"]
    br2 = params["roots2"]["b"].reshape(1, D)
    whs = params["feat_head"]["w"]
    bhs = params["feat_head"]["b"].reshape(1, NS)
    whr = params["role_head"]["w"]
    bhr = params["role_head"]["b"].reshape(1, NR)
    role_tbl = params["role_emb"]

    def per_graph(shape):
        nd = len(shape)
        return pl.BlockSpec((1,) + shape,
                            lambda b, nd=nd: (b,) + (0,) * nd)

    def whole(arr):
        nd = arr.ndim
        return pl.BlockSpec(arr.shape, lambda b, nd=nd: (0,) * nd)

    weights = [role_tbl, wt1, bt1, wt2, bt2, wsc, bsc, r1, br1, r2, br2,
               wm1ab, wsqzd, bm1, wm2, bm2,
               wcv1, bcv1, wc2t, bc2, wv2t, bv2,
               wn1, bn1, wn2, bn2,
               whs, bhs, whr, bhr]

    in_specs = ([per_graph((NP, 3)), per_graph((3, NP)), per_graph((NP, 3)),
                 per_graph((NP, NS)), per_graph((NP, NR)),
                 pl.BlockSpec((1, 1, 1), lambda b: (b, 0, 0))]
                + [whole(w) for w in weights])

    out_specs = [per_graph((NP, 3)), per_graph((NP, 3)),
                 per_graph((NP, NS)), per_graph((NP, NR))]
    out_shape = [jax.ShapeDtypeStruct((B, NP, 3), f32),
                 jax.ShapeDtypeStruct((B, NP, 3), f32),
                 jax.ShapeDtypeStruct((B, NP, NS), f32),
                 jax.ShapeDtypeStruct((B, NP, NR), f32)]

    xo, vo, so, ro = pl.pallas_call(
        functools.partial(_body, NP, D, L),
        grid=(B,),
        in_specs=in_specs,
        out_specs=out_specs,
        out_shape=out_shape,
        scratch_shapes=[pltpu.VMEM((K, NP, 1), f32)],
    )(xb, xt, vb, scb, roh, t2, *weights)

    return (xo.reshape(NTOT, 3), vo.reshape(NTOT, 3),
            so.reshape(NTOT, NS), ro.reshape(NTOT, NR))


# speedup vs baseline: 1.2218x; 1.2218x over previous
import functools

import jax
import jax.numpy as jnp
from jax import lax
from jax.experimental import pallas as pl
from jax.experimental.pallas import tpu as pltpu

K = 12


def _silu(z):
    return z * (1.0 / (1.0 + jnp.exp(-z)))


def _mm(a, b):
    return lax.dot_general(a, b, (((1,), (0,)), ((), ())),
                           preferred_element_type=jnp.float32)


def _mmT(a, b):
    return lax.dot_general(a, b, (((0,), (0,)), ((), ())),
                           preferred_element_type=jnp.float32)


def _body(NP, D, L,
          x3, xt, v3, sc, roh, t2, role_tbl,
          wt1, bt1, wt2, bt2, wsc, bsc, r1, br1, r2, br2,
          wm1, bm1, wm2, bm2, wc1, bc1, wc2t, bc2, wv1, bv1, wv2t, bv2,
          wn1, bn1, wn2, bn2, whs, bhs, whr, bhr,
          xo, vo, so, ro, idx_ref):
    f32 = jnp.float32
    xb = x3[0]
    xtb = xt[0]
    vb = v3[0]

    tval = t2[0, 0, 0]
    te = _silu(tval * wt1[...] + bt1[...])
    temb = _mm(te, wt2[...]) + bt2[...]
    scemb = _mm(sc[0], wsc[...]) + bsc[...]
    remb = _mm(roh[0], role_tbl[...])
    r1v = r1[...]
    pre = (_mm(scemb, r1v[0:D]) + _mm(remb, r1v[D:2 * D])
           + _mm(temb, r1v[2 * D:3 * D]) + br1[...])
    h0 = _mm(_silu(pre), r2[...]) + br2[...]

    colf = lax.broadcasted_iota(jnp.int32, (NP, NP), 1).astype(f32)
    rowf = lax.broadcasted_iota(jnp.int32, (NP, NP), 0).astype(f32)
    d2 = (((xb[:, 0:1] - xtb[0:1, :]) ** 2
           + (xb[:, 1:2] - xtb[1:2, :]) ** 2)
          + (xb[:, 2:3] - xtb[2:3, :]) ** 2)
    d2 = jnp.where(rowf == colf, 1e10, d2)
    for k in range(K):
        m = jnp.min(d2, axis=1, keepdims=True)
        idx = jnp.min(jnp.where(d2 == m, colf, f32(NP)), axis=1,
                      keepdims=True)
        idx_ref[k] = idx
        d2 = jnp.where(colf == idx, 1e10, d2)

    def layer(l, hxv):
        h, xc, vc = hxv
        wm1l = wm1[l]
        hw1a = _mm(h, wm1l[0:D])
        base = _mm(h, wm1l[D:2 * D]) + bm1[l]
        wsq = wm1l[2 * D:2 * D + 1]
        wzd = wm1l[2 * D + 1:2 * D + 2]
        wm2l, bm2l = wm2[l], bm2[l]
        wc1l, bc1l, wc2l, bc2l = wc1[l], bc1[l], wc2t[l], bc2[l]
        wv1l, bv1l, wv2l, bv2l = wv1[l], bv1[l], wv2t[l], bv2[l]

        def edge_slot(k, accs):
            aggr, acc2 = accs
            idx = idx_ref[k]
            sel = (colf == idx).astype(f32)
            rel = _mm(sel, xc) - xc
            sq = jnp.sum(rel * rel, axis=1, keepdims=True)
            zd = rel[:, 2:3]
            pre1 = _mm(sel, hw1a) + base + sq * wsq + zd * wzd
            msg = _silu(_mm(_silu(pre1), wm2l) + bm2l)
            tc = _silu(_mm(msg, wc1l) + bc1l)
            xw = (jnp.sum(tc * wc2l, axis=1, keepdims=True) + bc2l) \
                / (sq + 1e-8)
            tv = _silu(_mm(msg, wv1l) + bv1l)
            vw = (jnp.sum(tv * wv2l, axis=1, keepdims=True) + bv2l) \
                / (sq + 1e-8)
            aggr = aggr + _mmT(sel, msg)
            small = jnp.concatenate([rel * xw, rel * vw], axis=1)
            acc2 = acc2 + _mmT(sel, small)
            return aggr, acc2

        accs = (jnp.zeros((NP, D), f32), jnp.zeros((NP, 6), f32))
        for k in range(K):
            accs = edge_slot(k, accs)
        aggr, acc2 = accs
        xn = xc + acc2[:, 0:3]
        vn = vc + acc2[:, 3:6]
        wn1l = wn1[l]
        hpre = _silu(_mm(h, wn1l[0:D]) + _mm(aggr, wn1l[D:2 * D]) + bn1[l])
        hn = h + _mm(hpre, wn2[l]) + bn2[l]
        return hn, xn, vn

    h, xc, vc = lax.fori_loop(0, L, layer, (h0, xb, vb))

    xo[0] = xc
    vo[0] = vc
    so[0] = _mm(h, whs[...]) + bhs[...]
    ro[0] = _mm(h, whr[...]) + bhr[...]


def kernel(x, v, scalars, roles, t, batch, params):
    f32 = jnp.float32
    B = t.shape[0]
    NTOT = x.shape[0]
    NP = NTOT // B
    NR, D = params["role_emb"].shape
    NS = scalars.shape[1]
    lys = params["layers"]
    L = len(lys)

    def st_w(name):
        return jnp.stack([ly[name]["w"] for ly in lys])

    def st_b(name):
        return jnp.stack([ly[name]["b"] for ly in lys])[:, None, :]

    wm1, bm1 = st_w("msg1"), st_b("msg1")
    wm2, bm2 = st_w("msg2"), st_b("msg2")
    wc1, bc1 = st_w("coord1"), st_b("coord1")
    wv1, bv1 = st_w("vec1"), st_b("vec1")
    wn1, bn1 = st_w("node1"), st_b("node1")
    wn2, bn2 = st_w("node2"), st_b("node2")
    wc2t = jnp.stack([ly["coord2"]["w"].T for ly in lys])
    bc2 = jnp.stack([ly["coord2"]["b"] for ly in lys])[:, :, None]
    wv2t = jnp.stack([ly["vec2"]["w"].T for ly in lys])
    bv2 = jnp.stack([ly["vec2"]["b"] for ly in lys])[:, :, None]

    xb = x.reshape(B, NP, 3)
    xt = xb.transpose(0, 2, 1)
    vb = v.reshape(B, NP, 3)
    scb = scalars.reshape(B, NP, NS)
    roh = (roles.reshape(B, NP)[..., None]
           == jnp.arange(NR, dtype=roles.dtype)).astype(f32)
    t2 = t.reshape(B, 1, 1)

    wt1 = params["time1"]["w"]
    bt1 = params["time1"]["b"].reshape(1, D)
    wt2 = params["time2"]["w"]
    bt2 = params["time2"]["b"].reshape(1, D)
    wsc = params["scalar_emb"]["w"]
    bsc = params["scalar_emb"]["b"].reshape(1, D)
    r1 = params["roots1"]["w"]
    br1 = params["roots1"]["b"].reshape(1, D)
    r2 = params["roots2"]["w"]
    br2 = params["roots2"]["b"].reshape(1, D)
    whs = params["feat_head"]["w"]
    bhs = params["feat_head"]["b"].reshape(1, NS)
    whr = params["role_head"]["w"]
    bhr = params["role_head"]["b"].reshape(1, NR)
    role_tbl = params["role_emb"]

    def per_graph(shape):
        nd = len(shape)
        return pl.BlockSpec((1,) + shape,
                            lambda b, nd=nd: (b,) + (0,) * nd)

    def whole(arr):
        nd = arr.ndim
        return pl.BlockSpec(arr.shape, lambda b, nd=nd: (0,) * nd)

    weights = [role_tbl, wt1, bt1, wt2, bt2, wsc, bsc, r1, br1, r2, br2,
               wm1, bm1, wm2, bm2, wc1, bc1, wc2t, bc2, wv1, bv1, wv2t,
               bv2, wn1, bn1, wn2, bn2, whs, bhs, whr, bhr]

    in_specs = ([per_graph((NP, 3)), per_graph((3, NP)), per_graph((NP, 3)),
                 per_graph((NP, NS)), per_graph((NP, NR)),
                 pl.BlockSpec((1, 1, 1), lambda b: (b, 0, 0))]
                + [whole(w) for w in weights])

    out_specs = [per_graph((NP, 3)), per_graph((NP, 3)),
                 per_graph((NP, NS)), per_graph((NP, NR))]
    out_shape = [jax.ShapeDtypeStruct((B, NP, 3), f32),
                 jax.ShapeDtypeStruct((B, NP, 3), f32),
                 jax.ShapeDtypeStruct((B, NP, NS), f32),
                 jax.ShapeDtypeStruct((B, NP, NR), f32)]

    xo, vo, so, ro = pl.pallas_call(
        functools.partial(_body, NP, D, L),
        grid=(B,),
        in_specs=in_specs,
        out_specs=out_specs,
        out_shape=out_shape,
        scratch_shapes=[pltpu.VMEM((K, NP, 1), f32)],
    )(xb, xt, vb, scb, roh, t2, *weights)

    return (xo.reshape(NTOT, 3), vo.reshape(NTOT, 3),
            so.reshape(NTOT, NS), ro.reshape(NTOT, NR))


# speedup vs baseline: 1.9975x; 1.6349x over previous
import functools

import jax
import jax.numpy as jnp
from jax import lax
from jax.experimental import pallas as pl
from jax.experimental.pallas import tpu as pltpu

K = 12


def _silu(z):
    return z * (1.0 / (1.0 + jnp.exp(-z)))


def _mm(a, b):
    return lax.dot_general(a, b, (((1,), (0,)), ((), ())),
                           preferred_element_type=jnp.float32)


def _mmT(a, b):
    return lax.dot_general(a, b, (((0,), (0,)), ((), ())),
                           preferred_element_type=jnp.float32)


def _body(NP, D, L,
          x3, xt, v3, sc, roh, t2, role_tbl,
          wt1, bt1, wt2, bt2, wsc, bsc, r1, br1, r2, br2,
          wm1, bm1, wm2, bm2, wc1, bc1, wc2t, bc2, wv1, bv1, wv2t, bv2,
          wn1, bn1, wn2, bn2, whs, bhs, whr, bhr,
          xo, vo, so, ro, idx_ref):
    f32 = jnp.float32
    xb = x3[0]
    xtb = xt[0]
    vb = v3[0]

    tval = t2[0, 0, 0]
    te = _silu(tval * wt1[...] + bt1[...])
    temb = _mm(te, wt2[...]) + bt2[...]
    scemb = _mm(sc[0], wsc[...]) + bsc[...]
    remb = _mm(roh[0], role_tbl[...])
    r1v = r1[...]
    pre = (_mm(scemb, r1v[0:D]) + _mm(remb, r1v[D:2 * D])
           + _mm(temb, r1v[2 * D:3 * D]) + br1[...])
    h0 = _mm(_silu(pre), r2[...]) + br2[...]

    colf = lax.broadcasted_iota(jnp.int32, (NP, NP), 1).astype(f32)
    rowf = lax.broadcasted_iota(jnp.int32, (NP, NP), 0).astype(f32)
    d2 = (((xb[:, 0:1] - xtb[0:1, :]) ** 2
           + (xb[:, 1:2] - xtb[1:2, :]) ** 2)
          + (xb[:, 2:3] - xtb[2:3, :]) ** 2)
    d2 = jnp.where(rowf == colf, 1e10, d2)
    for k in range(K):
        m = jnp.min(d2, axis=1, keepdims=True)
        idx = jnp.min(jnp.where(d2 == m, colf, f32(NP)), axis=1,
                      keepdims=True)
        idx_ref[k] = idx
        d2 = jnp.where(colf == idx, 1e10, d2)

    def layer(l, hxv):
        h, xc, vc = hxv
        wm1l = wm1[l]
        hw1a = _mm(h, wm1l[0:D])
        base = _mm(h, wm1l[D:2 * D]) + bm1[l]
        wsq = wm1l[2 * D:2 * D + 1]
        wzd = wm1l[2 * D + 1:2 * D + 2]
        wm2l, bm2l = wm2[l], bm2[l]
        wc1l, bc1l, wc2l, bc2l = wc1[l], bc1[l], wc2t[l], bc2[l]
        wv1l, bv1l, wv2l, bv2l = wv1[l], bv1[l], wv2t[l], bv2[l]

        def edge_slot(k, accs):
            aggr, acc2 = accs
            idx = idx_ref[k]
            sel = (colf == idx).astype(f32)
            selt = (rowf == idx.reshape(1, NP)).astype(f32)
            rel = _mm(sel, xc) - xc
            sq = jnp.sum(rel * rel, axis=1, keepdims=True)
            zd = rel[:, 2:3]
            pre1 = _mm(sel, hw1a) + base + sq * wsq + zd * wzd
            msg = _silu(_mm(_silu(pre1), wm2l) + bm2l)
            tc = _silu(_mm(msg, wc1l) + bc1l)
            xw = (jnp.sum(tc * wc2l, axis=1, keepdims=True) + bc2l) \
                / (sq + 1e-8)
            tv = _silu(_mm(msg, wv1l) + bv1l)
            vw = (jnp.sum(tv * wv2l, axis=1, keepdims=True) + bv2l) \
                / (sq + 1e-8)
            aggr = aggr + _mm(selt, msg)
            small = jnp.concatenate([rel * xw, rel * vw], axis=1)
            acc2 = acc2 + _mm(selt, small)
            return aggr, acc2

        accs = (jnp.zeros((NP, D), f32), jnp.zeros((NP, 6), f32))
        for k in range(K):
            accs = edge_slot(k, accs)
        aggr, acc2 = accs
        xn = xc + acc2[:, 0:3]
        vn = vc + acc2[:, 3:6]
        wn1l = wn1[l]
        hpre = _silu(_mm(h, wn1l[0:D]) + _mm(aggr, wn1l[D:2 * D]) + bn1[l])
        hn = h + _mm(hpre, wn2[l]) + bn2[l]
        return hn, xn, vn

    h, xc, vc = lax.fori_loop(0, L, layer, (h0, xb, vb))

    xo[0] = xc
    vo[0] = vc
    so[0] = _mm(h, whs[...]) + bhs[...]
    ro[0] = _mm(h, whr[...]) + bhr[...]


def kernel(x, v, scalars, roles, t, batch, params):
    f32 = jnp.float32
    B = t.shape[0]
    NTOT = x.shape[0]
    NP = NTOT // B
    NR, D = params["role_emb"].shape
    NS = scalars.shape[1]
    lys = params["layers"]
    L = len(lys)

    def st_w(name):
        return jnp.stack([ly[name]["w"] for ly in lys])

    def st_b(name):
        return jnp.stack([ly[name]["b"] for ly in lys])[:, None, :]

    wm1, bm1 = st_w("msg1"), st_b("msg1")
    wm2, bm2 = st_w("msg2"), st_b("msg2")
    wc1, bc1 = st_w("coord1"), st_b("coord1")
    wv1, bv1 = st_w("vec1"), st_b("vec1")
    wn1, bn1 = st_w("node1"), st_b("node1")
    wn2, bn2 = st_w("node2"), st_b("node2")
    wc2t = jnp.stack([ly["coord2"]["w"].T for ly in lys])
    bc2 = jnp.stack([ly["coord2"]["b"] for ly in lys])[:, :, None]
    wv2t = jnp.stack([ly["vec2"]["w"].T for ly in lys])
    bv2 = jnp.stack([ly["vec2"]["b"] for ly in lys])[:, :, None]

    xb = x.reshape(B, NP, 3)
    xt = xb.transpose(0, 2, 1)
    vb = v.reshape(B, NP, 3)
    scb = scalars.reshape(B, NP, NS)
    roh = (roles.reshape(B, NP)[..., None]
           == jnp.arange(NR, dtype=roles.dtype)).astype(f32)
    t2 = t.reshape(B, 1, 1)

    wt1 = params["time1"]["w"]
    bt1 = params["time1"]["b"].reshape(1, D)
    wt2 = params["time2"]["w"]
    bt2 = params["time2"]["b"].reshape(1, D)
    wsc = params["scalar_emb"]["w"]
    bsc = params["scalar_emb"]["b"].reshape(1, D)
    r1 = params["roots1"]["w"]
    br1 = params["roots1"]["b"].reshape(1, D)
    r2 = params["roots2"]["w"]
    br2 = params["roots2"]["b"].reshape(1, D)
    whs = params["feat_head"]["w"]
    bhs = params["feat_head"]["b"].reshape(1, NS)
    whr = params["role_head"]["w"]
    bhr = params["role_head"]["b"].reshape(1, NR)
    role_tbl = params["role_emb"]

    def per_graph(shape):
        nd = len(shape)
        return pl.BlockSpec((1,) + shape,
                            lambda b, nd=nd: (b,) + (0,) * nd)

    def whole(arr):
        nd = arr.ndim
        return pl.BlockSpec(arr.shape, lambda b, nd=nd: (0,) * nd)

    weights = [role_tbl, wt1, bt1, wt2, bt2, wsc, bsc, r1, br1, r2, br2,
               wm1, bm1, wm2, bm2, wc1, bc1, wc2t, bc2, wv1, bv1, wv2t,
               bv2, wn1, bn1, wn2, bn2, whs, bhs, whr, bhr]

    in_specs = ([per_graph((NP, 3)), per_graph((3, NP)), per_graph((NP, 3)),
                 per_graph((NP, NS)), per_graph((NP, NR)),
                 pl.BlockSpec((1, 1, 1), lambda b: (b, 0, 0))]
                + [whole(w) for w in weights])

    out_specs = [per_graph((NP, 3)), per_graph((NP, 3)),
                 per_graph((NP, NS)), per_graph((NP, NR))]
    out_shape = [jax.ShapeDtypeStruct((B, NP, 3), f32),
                 jax.ShapeDtypeStruct((B, NP, 3), f32),
                 jax.ShapeDtypeStruct((B, NP, NS), f32),
                 jax.ShapeDtypeStruct((B, NP, NR), f32)]

    xo, vo, so, ro = pl.pallas_call(
        functools.partial(_body, NP, D, L),
        grid=(B,),
        in_specs=in_specs,
        out_specs=out_specs,
        out_shape=out_shape,
        scratch_shapes=[pltpu.VMEM((K, NP, 1), f32)],
    )(xb, xt, vb, scb, roh, t2, *weights)

    return (xo.reshape(NTOT, 3), vo.reshape(NTOT, 3),
            so.reshape(NTOT, NS), ro.reshape(NTOT, NR))
